# baseline (speedup 1.0000x reference)
import functools

import jax
import jax.numpy as jnp
from jax import lax
from jax.experimental import pallas as pl
from jax.experimental.pallas import tpu as pltpu
from jax.experimental.pallas import tpu_sc as plsc

_NW = 32
_RPW = 16384 // _NW
_XR = 2048 // 16
_ZR = 2048 // _NW


def _body(x_hbm, y_hbm, z_hbm, xo_hbm, yo_hbm, zo_hbm, strip_v, out_v, sem):
    c = lax.axis_index("c")
    s = lax.axis_index("s")
    w = s * 2 + c

    p = w // 16
    src_row = jnp.where(p == 0, 8, 5)
    r0 = (w % 16) * _XR
    x_dma = pltpu.async_copy(
        x_hbm.at[src_row, pl.ds(r0, _XR)], xo_hbm.at[p, pl.ds(r0, _XR)], sem
    )
    z_dma = pltpu.async_copy(
        z_hbm.at[2, pl.ds(w * _ZR, _ZR)], zo_hbm.at[0, pl.ds(w * _ZR, _ZR)], sem
    )

    pltpu.sync_copy(y_hbm.at[pl.ds(w * _RPW, _RPW), pl.ds(0, 128)], strip_v)
    lanes = lax.iota(jnp.int32, 16)
    for j in range(_RPW * 2 // 16):
        k16 = j * 16 + lanes
        out_v[j] = plsc.load_gather(strip_v, [k16 >> 1, 1 - (k16 & 1)])
    pltpu.sync_copy(out_v, yo_hbm.at[w])

    x_dma.wait()
    z_dma.wait()


def kernel(x, y, z):
    x2 = x.reshape(16, 2048, 1024)
    y2 = y.reshape(16384, 2048)
    z2 = z.reshape(8, 2048, 1024)

    mesh = plsc.VectorSubcoreMesh(core_axis_name="c", subcore_axis_name="s")
    run = functools.partial(
        pl.kernel,
        mesh=mesh,
        out_type=(
            jax.ShapeDtypeStruct((2, 2048, 1024), jnp.float32),
            jax.ShapeDtypeStruct((_NW, _RPW * 2 // 16, 16), jnp.float32),
            jax.ShapeDtypeStruct((1, 2048, 1024), jnp.float32),
        ),
        scratch_types=[
            pltpu.VMEM((_RPW, 128), jnp.float32),
            pltpu.VMEM((_RPW * 2 // 16, 16), jnp.float32),
            pltpu.SemaphoreType.DMA,
        ],
        compiler_params=pltpu.CompilerParams(needs_layout_passes=False),
    )(_body)
    x_out, y_out, z_out = run(x2, y2, z2)
    return (x_out, y_out.reshape(4, 4096, 2), z_out)


# speedup vs baseline: 1.0317x; 1.0317x over previous
import jax
import jax.numpy as jnp
from jax.experimental import pallas as pl
from jax.experimental.pallas import tpu as pltpu

_G = 8
_YR = 4096 // _G


def _copies(x_ref, z_ref, xo_ref, zo_ref, sem):
    return (
        pltpu.make_async_copy(x_ref.at[2, 0], xo_ref.at[0], sem),
        pltpu.make_async_copy(x_ref.at[1, 1], xo_ref.at[1], sem),
        pltpu.make_async_copy(z_ref.at[0, 2], zo_ref.at[0], sem),
    )


def _body(x_ref, z_ref, y_ref, xo_ref, zo_ref, yo_ref, sem):
    g = pl.program_id(0)

    @pl.when(g == 0)
    def _start():
        for c in _copies(x_ref, z_ref, xo_ref, zo_ref, sem):
            c.start()

    yo_ref[:, :, 0] = y_ref[:, :, 1]
    yo_ref[:, :, 1] = y_ref[:, :, 0]

    @pl.when(g == _G - 1)
    def _drain():
        for c in _copies(x_ref, z_ref, xo_ref, zo_ref, sem):
            c.wait()


def kernel(x, y, z):
    out_shapes = (
        jax.ShapeDtypeStruct((2, 2048, 1024), jnp.float32),
        jax.ShapeDtypeStruct((1, 2048, 1024), jnp.float32),
        jax.ShapeDtypeStruct((4, 4096, 2), jnp.float32),
    )
    in_specs = [
        pl.BlockSpec(memory_space=pl.ANY),
        pl.BlockSpec(memory_space=pl.ANY),
        pl.BlockSpec((4, _YR, 128), lambda g: (0, g, 0)),
    ]
    out_specs = (
        pl.BlockSpec(memory_space=pl.ANY),
        pl.BlockSpec(memory_space=pl.ANY),
        pl.BlockSpec((4, _YR, 2), lambda g: (0, g, 0)),
    )
    x_out, z_out, y_out = pl.pallas_call(
        _body,
        grid=(_G,),
        in_specs=in_specs,
        out_specs=out_specs,
        out_shape=out_shapes,
        scratch_shapes=[pltpu.SemaphoreType.DMA],
    )(x, z, y)
    return (x_out, y_out, z_out)


# speedup vs baseline: 27.6103x; 26.7627x over previous
import jax
import jax.numpy as jnp
from jax.experimental import pallas as pl

_R = 512
_G = 2048 // _R
_YR = 4096 // _G


def _body(xa_ref, xb_ref, z_ref, y_ref, xo_ref, yo_ref, zo_ref):
    xo_ref[0] = xa_ref[0, 0]
    xo_ref[1] = xb_ref[0, 0]
    zo_ref[0] = z_ref[0, 0]
    yo_ref[:, :, 0] = y_ref[:, :, 1]
    yo_ref[:, :, 1] = y_ref[:, :, 0]


def kernel(x, y, z):
    out_shapes = (
        jax.ShapeDtypeStruct((2, 2048, 1024), jnp.float32),
        jax.ShapeDtypeStruct((4, 4096, 2), jnp.float32),
        jax.ShapeDtypeStruct((1, 2048, 1024), jnp.float32),
    )
    in_specs = [
        pl.BlockSpec((1, 1, _R, 1024), lambda g: (2, 0, g, 0)),
        pl.BlockSpec((1, 1, _R, 1024), lambda g: (1, 1, g, 0)),
        pl.BlockSpec((1, 1, _R, 1024), lambda g: (0, 2, g, 0)),
        pl.BlockSpec((4, _YR, 128), lambda g: (0, g, 0)),
    ]
    out_specs = (
        pl.BlockSpec((2, _R, 1024), lambda g: (0, g, 0)),
        pl.BlockSpec((4, _YR, 2), lambda g: (0, g, 0)),
        pl.BlockSpec((1, _R, 1024), lambda g: (0, g, 0)),
    )
    return pl.pallas_call(
        _body,
        grid=(_G,),
        in_specs=in_specs,
        out_specs=out_specs,
        out_shape=out_shapes,
    )(x, x, z, y)
